# Initial kernel scaffold; baseline (speedup 1.0000x reference)
import jax
import jax.numpy as jnp
from jax.experimental import pallas as pl


def kernel(x, emb_weight):
    raise NotImplementedError("write your pallas kernel here")


# speedup vs baseline: 2.1884x; 2.1884x over previous
import functools

import jax
import jax.numpy as jnp
from jax import lax
from jax.experimental import pallas as pl
from jax.experimental.pallas import tpu as pltpu
from jax.experimental.pallas import tpu_sc as plsc

SEQ_LEN = 8192
MODEL_DIM = 1024
N_IDX = 4 * 8192

_info = plsc.get_sparse_core_info()
_NC = _info.num_cores
_NS = _info.num_subcores
_NW = _NC * _NS
_B_PER_W = N_IDX // _NW
_CHUNK = 64
_NCHUNK = _B_PER_W // _CHUNK


def _make_gather():
  mesh = plsc.VectorSubcoreMesh(core_axis_name="c", subcore_axis_name="s")

  @functools.partial(
      pl.kernel,
      mesh=mesh,
      out_type=jax.ShapeDtypeStruct((N_IDX, MODEL_DIM), jnp.float32),
      scratch_types=[
          pltpu.VMEM((_B_PER_W,), jnp.int32),
          pltpu.VMEM((_CHUNK, MODEL_DIM), jnp.float32),
          pltpu.SemaphoreType.DMA,
      ],
  )
  def gather_kernel(table_hbm, idx_hbm, out_hbm, idx_v, buf, gsem):
    wid = lax.axis_index("s") * _NC + lax.axis_index("c")
    base = wid * _B_PER_W
    pltpu.sync_copy(idx_hbm.at[pl.ds(base, _B_PER_W)], idx_v)

    def chunk_body(c, carry):
      row = c * _CHUNK
      pltpu.async_copy(
          table_hbm.at[idx_v.at[pl.ds(row, _CHUNK)]], buf, gsem
      ).wait()
      pltpu.sync_copy(buf, out_hbm.at[pl.ds(base + row, _CHUNK)])
      return carry

    lax.fori_loop(0, _NCHUNK, chunk_body, 0)

  return gather_kernel


_gather = _make_gather()


def kernel(x, emb_weight):
  xf = x.reshape(-1).astype(jnp.int32)
  out = _gather(emb_weight, xf)
  return out.reshape(x.shape + (emb_weight.shape[1],))
